# baseline (speedup 1.0000x reference)
import functools

import jax
import jax.numpy as jnp
import numpy as np
from jax import lax
from jax.experimental import pallas as pl
from jax.experimental.pallas import tpu as pltpu
from jax.experimental.pallas import tpu_sc as plsc
from jax._src import config as _jax_config

N_NODES = 10000
N_EDGES = 320000
HIDDEN = 128
N_ITEMS = 100000
N_SESSIONS = 32

NC = 2
NS = 16
NW = NC * NS

NODE_PAD = 10240
GCH = 40
ECH = 96
NBUF = 3
EPW = N_EDGES // NW
ETAIL = EPW % ECH
ACC_ROWS = N_NODES
ROWS_PER_SUB = 624

BN = 400
BV = 2048

_f32 = jnp.float32


def _sc_embedding_gather(emb_weight, x_pad):
    with _jax_config.enable_x64(False):
        return _sc_embedding_gather_x32(emb_weight, x_pad)


def _sc_embedding_gather_x32(emb_weight, x2d):
    mesh = plsc.VectorSubcoreMesh(core_axis_name="c", subcore_axis_name="s")
    npw = NODE_PAD // NW
    nch = npw // GCH

    @functools.partial(
        pl.kernel,
        out_type=jax.ShapeDtypeStruct((NODE_PAD, HIDDEN), _f32),
        mesh=mesh,
        scratch_types=[
            pltpu.VMEM((nch, GCH), jnp.int32),
            pltpu.VMEM((npw, HIDDEN), _f32),
            pltpu.SemaphoreType.DMA,
        ],
    )
    def k(emb_hbm, idx_hbm, out_hbm, idx_v, rows_v, sem):
        wid = lax.axis_index("s") * jnp.int32(NC) + lax.axis_index("c")
        pltpu.sync_copy(idx_hbm.at[pl.ds(wid * jnp.int32(nch), nch)], idx_v)
        for j in range(nch):
            pltpu.async_copy(emb_hbm.at[idx_v.at[jnp.int32(j)]],
                             rows_v.at[pl.ds(jnp.int32(j * GCH), GCH)], sem)
        for j in range(nch):
            pltpu.make_async_copy(emb_hbm.at[pl.ds(jnp.int32(0), GCH)],
                                  rows_v.at[pl.ds(jnp.int32(0), GCH)], sem).wait()
        pltpu.sync_copy(rows_v, out_hbm.at[pl.ds(wid * jnp.int32(npw), npw)])

    return k(emb_weight, x2d)


def _sc_edge_scatter(emb, src, dst, zeros):
    with _jax_config.enable_x64(False):
        return _sc_edge_scatter_x32(emb, src, dst, zeros)


def _sc_edge_scatter_x32(emb, src2d, dst2d, zeros):
    mesh = plsc.VectorSubcoreMesh(core_axis_name="c", subcore_axis_name="s")
    nch = EPW // ECH
    nmain = nch - (nch % NBUF)

    @functools.partial(
        pl.kernel,
        out_type=jax.ShapeDtypeStruct((NC * N_NODES, HIDDEN), _f32),
        mesh=mesh,
        scratch_types=(
            [pltpu.VMEM((ECH,), jnp.int32)] * (2 * NBUF)
            + [pltpu.VMEM((ETAIL,), jnp.int32)] * 2
            + [pltpu.VMEM((ECH, HIDDEN), _f32)] * NBUF
            + [pltpu.VMEM_SHARED((ACC_ROWS, HIDDEN), _f32)]
            + [pltpu.SemaphoreType.DMA] * NBUF
        ),
    )
    def k(emb_hbm, src_hbm, dst_hbm, zeros_hbm, m_hbm,
          src_v0, src_v1, src_v2, dst_v0, dst_v1, dst_v2, src_t, dst_t,
          rows0, rows1, rows2, acc, sem0, sem1, sem2):
        cid = lax.axis_index("c")
        sid = lax.axis_index("s")
        wid = sid * jnp.int32(NC) + cid
        off = sid * jnp.int32(ROWS_PER_SUB)
        srcs = (src_v0, src_v1, src_v2)
        dsts = (dst_v0, dst_v1, dst_v2)
        rows = (rows0, rows1, rows2)
        sems = (sem0, sem1, sem2)

        pltpu.sync_copy(zeros_hbm.at[pl.ds(0, ROWS_PER_SUB)],
                        acc.at[pl.ds(off, ROWS_PER_SUB)])

        @pl.when(sid == jnp.int32(NS - 1))
        def _():
            pltpu.sync_copy(zeros_hbm.at[pl.ds(ROWS_PER_SUB, 16)],
                            acc.at[pl.ds(NS * ROWS_PER_SUB, 16)])

        plsc.subcore_barrier()

        def stage_and_gather(ch, b):
            base = wid * jnp.int32(EPW) + ch * jnp.int32(ECH)
            pltpu.sync_copy(src_hbm.at[pl.ds(base, ECH)], srcs[b])
            pltpu.sync_copy(dst_hbm.at[pl.ds(base, ECH)], dsts[b])
            pltpu.async_copy(emb_hbm.at[srcs[b]], rows[b], sems[b])

        def wait_gather(b):
            pltpu.make_async_copy(emb_hbm.at[pl.ds(jnp.int32(0), ECH)],
                                  rows[b], sems[b]).wait()

        for b in range(NBUF):
            stage_and_gather(jnp.int32(b), b)

        @pl.loop(np.int32(0), np.int32(nmain), step=np.int32(NBUF))
        def _(g):
            for b in range(NBUF):
                ch = g + jnp.int32(b)
                wait_gather(b)
                pltpu.sync_copy(rows[b], acc.at[dsts[b]], add=True)
                nxt = ch + jnp.int32(NBUF)

                @pl.when(nxt < jnp.int32(nch))
                def _():
                    stage_and_gather(nxt, b)

        tbase = wid * jnp.int32(EPW) + jnp.int32(nch * ECH)
        pltpu.sync_copy(src_hbm.at[pl.ds(tbase, ETAIL)], src_t)
        pltpu.sync_copy(dst_hbm.at[pl.ds(tbase, ETAIL)], dst_t)
        for b in range(nch - nmain):
            wait_gather(b)
            pltpu.sync_copy(rows[b], acc.at[dsts[b]], add=True)
        pltpu.async_copy(emb_hbm.at[src_t],
                         rows0.at[pl.ds(jnp.int32(0), ETAIL)], sem0).wait()
        pltpu.sync_copy(rows0.at[pl.ds(jnp.int32(0), ETAIL)],
                        acc.at[dst_t], add=True)
        plsc.subcore_barrier()

        obase = cid * jnp.int32(N_NODES)
        pltpu.sync_copy(acc.at[pl.ds(off, ROWS_PER_SUB)],
                        m_hbm.at[pl.ds(obase + off, ROWS_PER_SUB)])

        @pl.when(sid == jnp.int32(NS - 1))
        def _():
            pltpu.sync_copy(acc.at[pl.ds(NS * ROWS_PER_SUB, 16)],
                            m_hbm.at[pl.ds(obase + jnp.int32(NS * ROWS_PER_SUB), 16)])

    return k(emb, src2d, dst2d, zeros)


def _dotT(a, b):
    return lax.dot_general(a, b, (((1,), (1,)), ((), ())),
                           preferred_element_type=_f32)


def _node_body(ma_ref, mb_ref, emb_ref, wih_ref, whh_ref,
               batch_ref, bnext_ref, b0_ref, w1_ref, w2_ref, b2_ref,
               qw_ref, qb_ref, vn_ref, sg_ref, vi_s):
    p = pl.program_id(0)
    i = pl.program_id(1)
    b = batch_ref[...]
    sess = lax.broadcasted_iota(jnp.int32, (BN, N_SESSIONS), 1)

    @pl.when(p == 0)
    def _():
        m = ma_ref[...] + mb_ref[...]
        h = emb_ref[...]
        gi = _dotT(m, wih_ref[...])
        gh = _dotT(h, whh_ref[...])
        r = jax.nn.sigmoid(gi[:, :HIDDEN] + gh[:, :HIDDEN])
        z = jax.nn.sigmoid(gi[:, HIDDEN:2 * HIDDEN] + gh[:, HIDDEN:2 * HIDDEN])
        n = jnp.tanh(gi[:, 2 * HIDDEN:] + r * gh[:, 2 * HIDDEN:])
        v = (1.0 - z) * n + z * h
        vi_s[pl.ds(i * BN, BN), :] = v

        bn = bnext_ref[...]
        b0 = b0_ref[...]
        last = (b <= sess) & (bn > sess)
        i_glob = i * BN + lax.broadcasted_iota(jnp.int32, (BN, N_SESSIONS), 0)
        wrap = (i_glob == N_NODES - 1) & (sess < b0)
        onehot_last = (last | wrap).astype(_f32)
        contrib = lax.dot_general(onehot_last, v, (((0,), (0,)), ((), ())),
                                  preferred_element_type=_f32)

        @pl.when(i == 0)
        def _():
            vn_ref[...] = jnp.zeros_like(vn_ref)

        vn_ref[...] += contrib

    @pl.when(p == 1)
    def _():
        v = vi_s[pl.ds(i * BN, BN), :]
        onehot = (b == sess).astype(_f32)
        vn_rep = lax.dot_general(onehot, vn_ref[...], (((1,), (0,)), ((), ())),
                                 preferred_element_type=_f32)
        q1 = _dotT(vn_rep, w1_ref[...])
        q2 = _dotT(v, w2_ref[...]) + b2_ref[...]
        a = jax.nn.sigmoid(q1 + q2)
        alpha = jnp.sum(a * qw_ref[...], axis=1, keepdims=True) + qb_ref[0, 0]
        contrib = lax.dot_general(onehot, alpha * v, (((0,), (0,)), ((), ())),
                                  preferred_element_type=_f32)

        @pl.when(i == 0)
        def _():
            sg_ref[...] = jnp.zeros_like(sg_ref)

        sg_ref[...] += contrib


def _tc_node(ma, mb, emb, w_ih, w_hh, batch2, bnext2, b00, w1, w2, b2r, qwr, qbr):
    with _jax_config.enable_x64(False):
        return _tc_node_x32(ma, mb, emb, w_ih, w_hh, batch2, bnext2, b00,
                            w1, w2, b2r, qwr, qbr)


def _tc_node_x32(ma, mb, emb, w_ih, w_hh, batch2, bnext2, b00, w1, w2, b2r,
                 qwr, qbr):
    grid = (2, N_NODES // BN)
    blk0 = pl.BlockSpec((BN, HIDDEN), lambda p, i: (i * (1 - p), 0))
    full = lambda r, c: pl.BlockSpec((r, c), lambda p, i: (0, 0))
    sess_spec = pl.BlockSpec((N_SESSIONS, HIDDEN), lambda p, i: (0, 0))
    return pl.pallas_call(
        _node_body,
        grid=grid,
        in_specs=[
            blk0, blk0, blk0,
            full(3 * HIDDEN, HIDDEN), full(3 * HIDDEN, HIDDEN),
            pl.BlockSpec((BN, 1), lambda p, i: (i, 0)),
            pl.BlockSpec((BN, 1), lambda p, i: (i * (1 - p), 0)),
            full(1, 1),
            full(HIDDEN, HIDDEN), full(HIDDEN, HIDDEN),
            full(1, HIDDEN), full(1, HIDDEN), full(1, 1),
        ],
        out_specs=[sess_spec, sess_spec],
        out_shape=[
            jax.ShapeDtypeStruct((N_SESSIONS, HIDDEN), _f32),
            jax.ShapeDtypeStruct((N_SESSIONS, HIDDEN), _f32),
        ],
        scratch_shapes=[pltpu.VMEM((N_NODES, HIDDEN), _f32)],
    )(ma, mb, emb, w_ih, w_hh, batch2, bnext2, b00, w1, w2, b2r, qwr, qbr)


def _vocab_body(vn_ref, sg_ref, w3_ref, wc_ref, wt_ref, wo_ref,
                emb_ref, z1_ref, z2_ref, z3_ref, xs_ref):
    @pl.when(pl.program_id(0) == 0)
    def _():
        sh = (_dotT(vn_ref[...], w3_ref[:, :HIDDEN])
              + _dotT(sg_ref[...], w3_ref[:, HIDDEN:]))
        sig = jax.nn.sigmoid(sh)
        xs_ref[:N_SESSIONS, :] = _dotT(sig, wc_ref[...])
        xs_ref[N_SESSIONS:2 * N_SESSIONS, :] = _dotT(sig, wt_ref[...])
        xs_ref[2 * N_SESSIONS:, :] = _dotT(sig, wo_ref[...])

    zz = _dotT(xs_ref[...], emb_ref[...])
    z1_ref[...] = zz[:N_SESSIONS]
    z2_ref[...] = zz[N_SESSIONS:2 * N_SESSIONS]
    z3_ref[...] = zz[2 * N_SESSIONS:]


def _tc_vocab(vn, sg, w3, wc, wt, wo, emb_weight):
    with _jax_config.enable_x64(False):
        return _tc_vocab_x32(vn, sg, w3, wc, wt, wo, emb_weight)


def _tc_vocab_x32(vn, sg, w3, wc, wt, wo, emb_weight):
    grid = (pl.cdiv(N_ITEMS, BV),)
    full = lambda r, c: pl.BlockSpec((r, c), lambda i: (0, 0))
    return pl.pallas_call(
        _vocab_body,
        grid=grid,
        in_specs=[
            full(N_SESSIONS, HIDDEN), full(N_SESSIONS, HIDDEN),
            full(HIDDEN, 2 * HIDDEN),
            full(HIDDEN, HIDDEN), full(HIDDEN, HIDDEN), full(HIDDEN, HIDDEN),
            pl.BlockSpec((BV, HIDDEN), lambda i: (i, 0)),
        ],
        out_specs=[pl.BlockSpec((N_SESSIONS, BV), lambda i: (0, i))] * 3,
        out_shape=[jax.ShapeDtypeStruct((N_SESSIONS, N_ITEMS), _f32)] * 3,
        scratch_shapes=[pltpu.VMEM((3 * N_SESSIONS, HIDDEN), _f32)],
    )(vn, sg, w3, wc, wt, wo, emb_weight)


def kernel(x, edge_index, batch, emb_weight, gru_w_ih, gru_w_hh, w1, w2, b2,
           q_w, q_b, w3, w_clicks, w_carts, w_orders):
    x32 = x.astype(jnp.int32)
    src = edge_index[0].astype(jnp.int32)
    dst = edge_index[1].astype(jnp.int32)
    batch32 = batch.astype(jnp.int32)

    x2d = jnp.pad(x32, (0, NODE_PAD - N_NODES)).reshape(NODE_PAD // GCH, GCH)
    emb_pad = _sc_embedding_gather(emb_weight, x2d)
    emb = emb_pad[:N_NODES]

    zeros = jnp.zeros((ROWS_PER_SUB + 16, HIDDEN), _f32)
    m_part = _sc_edge_scatter(emb_pad, src, dst, zeros)

    batch2 = batch32.reshape(N_NODES, 1)
    bnext2 = jnp.concatenate(
        [batch32[1:], jnp.full((1,), N_SESSIONS, jnp.int32)]).reshape(N_NODES, 1)
    b00 = batch32[:1].reshape(1, 1)

    vn, sg = _tc_node(m_part[:N_NODES], m_part[N_NODES:], emb,
                      gru_w_ih, gru_w_hh, batch2, bnext2, b00,
                      w1, w2, b2.reshape(1, HIDDEN), q_w.reshape(1, HIDDEN),
                      q_b.reshape(1, 1))

    z1, z2, z3 = _tc_vocab(vn, sg, w3, w_clicks, w_carts, w_orders, emb_weight)
    return (z1, z2, z3)


# speedup vs baseline: 1.0634x; 1.0634x over previous
import functools

import jax
import jax.numpy as jnp
import numpy as np
from jax import lax
from jax.experimental import pallas as pl
from jax.experimental.pallas import tpu as pltpu
from jax.experimental.pallas import tpu_sc as plsc
from jax._src import config as _jax_config

N_NODES = 10000
N_EDGES = 320000
HIDDEN = 128
N_ITEMS = 100000
N_SESSIONS = 32

NC = 2
NS = 16
NW = NC * NS

NODE_PAD = 10240
GCH = 40
ECH = 128
NBUF = 2
EPW = N_EDGES // NW
ETAIL = EPW % ECH
ACC_ROWS = N_NODES
ROWS_PER_SUB = 624

BN = 400
BV = 2048

_f32 = jnp.float32


def _sc_embedding_gather(emb_weight, x_pad):
    with _jax_config.enable_x64(False):
        return _sc_embedding_gather_x32(emb_weight, x_pad)


def _sc_embedding_gather_x32(emb_weight, x2d):
    mesh = plsc.VectorSubcoreMesh(core_axis_name="c", subcore_axis_name="s")
    npw = NODE_PAD // NW
    nch = npw // GCH

    @functools.partial(
        pl.kernel,
        out_type=jax.ShapeDtypeStruct((NODE_PAD, HIDDEN), _f32),
        mesh=mesh,
        scratch_types=[
            pltpu.VMEM((nch, GCH), jnp.int32),
            pltpu.VMEM((npw, HIDDEN), _f32),
            pltpu.SemaphoreType.DMA,
        ],
    )
    def k(emb_hbm, idx_hbm, out_hbm, idx_v, rows_v, sem):
        wid = lax.axis_index("s") * jnp.int32(NC) + lax.axis_index("c")
        pltpu.sync_copy(idx_hbm.at[pl.ds(wid * jnp.int32(nch), nch)], idx_v)
        for j in range(nch):
            pltpu.async_copy(emb_hbm.at[idx_v.at[jnp.int32(j)]],
                             rows_v.at[pl.ds(jnp.int32(j * GCH), GCH)], sem)
        for j in range(nch):
            pltpu.make_async_copy(emb_hbm.at[pl.ds(jnp.int32(0), GCH)],
                                  rows_v.at[pl.ds(jnp.int32(0), GCH)], sem).wait()
        pltpu.sync_copy(rows_v, out_hbm.at[pl.ds(wid * jnp.int32(npw), npw)])

    return k(emb_weight, x2d)


def _sc_edge_scatter(emb, src, dst, zeros):
    with _jax_config.enable_x64(False):
        return _sc_edge_scatter_x32(emb, src, dst, zeros)


def _sc_edge_scatter_x32(emb, src2d, dst2d, zeros):
    mesh = plsc.VectorSubcoreMesh(core_axis_name="c", subcore_axis_name="s")
    nch = EPW // ECH
    nmain = nch - (nch % NBUF)

    @functools.partial(
        pl.kernel,
        out_type=jax.ShapeDtypeStruct((NC * N_NODES, HIDDEN), _f32),
        mesh=mesh,
        scratch_types=(
            [pltpu.VMEM((ECH,), jnp.int32)] * (2 * NBUF)
            + [pltpu.VMEM((ETAIL,), jnp.int32)] * 2
            + [pltpu.VMEM((ECH, HIDDEN), _f32)] * NBUF
            + [pltpu.VMEM_SHARED((ACC_ROWS, HIDDEN), _f32)]
            + [pltpu.SemaphoreType.DMA] * NBUF
        ),
    )
    def k(emb_hbm, src_hbm, dst_hbm, zeros_hbm, m_hbm,
          src_v0, src_v1, dst_v0, dst_v1, src_t, dst_t,
          rows0, rows1, acc, sem0, sem1):
        cid = lax.axis_index("c")
        sid = lax.axis_index("s")
        wid = sid * jnp.int32(NC) + cid
        off = sid * jnp.int32(ROWS_PER_SUB)
        srcs = (src_v0, src_v1)
        dsts = (dst_v0, dst_v1)
        rows = (rows0, rows1)
        sems = (sem0, sem1)

        pltpu.sync_copy(zeros_hbm.at[pl.ds(0, ROWS_PER_SUB)],
                        acc.at[pl.ds(off, ROWS_PER_SUB)])

        @pl.when(sid == jnp.int32(NS - 1))
        def _():
            pltpu.sync_copy(zeros_hbm.at[pl.ds(ROWS_PER_SUB, 16)],
                            acc.at[pl.ds(NS * ROWS_PER_SUB, 16)])

        plsc.subcore_barrier()

        def stage_and_gather(ch, b):
            base = wid * jnp.int32(EPW) + ch * jnp.int32(ECH)
            pltpu.sync_copy(src_hbm.at[pl.ds(base, ECH)], srcs[b])
            pltpu.sync_copy(dst_hbm.at[pl.ds(base, ECH)], dsts[b])
            pltpu.async_copy(emb_hbm.at[srcs[b]], rows[b], sems[b])

        def wait_gather(b):
            pltpu.make_async_copy(emb_hbm.at[pl.ds(jnp.int32(0), ECH)],
                                  rows[b], sems[b]).wait()

        for b in range(NBUF):
            stage_and_gather(jnp.int32(b), b)

        @pl.loop(np.int32(0), np.int32(nmain), step=np.int32(NBUF))
        def _(g):
            for b in range(NBUF):
                ch = g + jnp.int32(b)
                wait_gather(b)
                pltpu.sync_copy(rows[b], acc.at[dsts[b]], add=True)
                nxt = ch + jnp.int32(NBUF)

                @pl.when(nxt < jnp.int32(nch))
                def _():
                    stage_and_gather(nxt, b)

        tbase = wid * jnp.int32(EPW) + jnp.int32(nch * ECH)
        pltpu.sync_copy(src_hbm.at[pl.ds(tbase, ETAIL)], src_t)
        pltpu.sync_copy(dst_hbm.at[pl.ds(tbase, ETAIL)], dst_t)
        for b in range(nch - nmain):
            wait_gather(b)
            pltpu.sync_copy(rows[b], acc.at[dsts[b]], add=True)
        pltpu.async_copy(emb_hbm.at[src_t],
                         rows0.at[pl.ds(jnp.int32(0), ETAIL)], sem0).wait()
        pltpu.sync_copy(rows0.at[pl.ds(jnp.int32(0), ETAIL)],
                        acc.at[dst_t], add=True)
        plsc.subcore_barrier()

        obase = cid * jnp.int32(N_NODES)
        pltpu.sync_copy(acc.at[pl.ds(off, ROWS_PER_SUB)],
                        m_hbm.at[pl.ds(obase + off, ROWS_PER_SUB)])

        @pl.when(sid == jnp.int32(NS - 1))
        def _():
            pltpu.sync_copy(acc.at[pl.ds(NS * ROWS_PER_SUB, 16)],
                            m_hbm.at[pl.ds(obase + jnp.int32(NS * ROWS_PER_SUB), 16)])

    return k(emb, src2d, dst2d, zeros)


def _dotT(a, b):
    return lax.dot_general(a, b, (((1,), (1,)), ((), ())),
                           preferred_element_type=_f32)


def _node_body(ma_ref, mb_ref, emb_ref, wih_ref, whh_ref,
               batch_ref, bnext_ref, b0_ref, w1_ref, w2_ref, b2_ref,
               qw_ref, qb_ref, vn_ref, sg_ref, vi_s):
    p = pl.program_id(0)
    i = pl.program_id(1)
    b = batch_ref[...]
    sess = lax.broadcasted_iota(jnp.int32, (BN, N_SESSIONS), 1)

    @pl.when(p == 0)
    def _():
        m = ma_ref[...] + mb_ref[...]
        h = emb_ref[...]
        gi = _dotT(m, wih_ref[...])
        gh = _dotT(h, whh_ref[...])
        r = jax.nn.sigmoid(gi[:, :HIDDEN] + gh[:, :HIDDEN])
        z = jax.nn.sigmoid(gi[:, HIDDEN:2 * HIDDEN] + gh[:, HIDDEN:2 * HIDDEN])
        n = jnp.tanh(gi[:, 2 * HIDDEN:] + r * gh[:, 2 * HIDDEN:])
        v = (1.0 - z) * n + z * h
        vi_s[pl.ds(i * BN, BN), :] = v

        bn = bnext_ref[...]
        b0 = b0_ref[...]
        last = (b <= sess) & (bn > sess)
        i_glob = i * BN + lax.broadcasted_iota(jnp.int32, (BN, N_SESSIONS), 0)
        wrap = (i_glob == N_NODES - 1) & (sess < b0)
        onehot_last = (last | wrap).astype(_f32)
        contrib = lax.dot_general(onehot_last, v, (((0,), (0,)), ((), ())),
                                  preferred_element_type=_f32)

        @pl.when(i == 0)
        def _():
            vn_ref[...] = jnp.zeros_like(vn_ref)

        vn_ref[...] += contrib

    @pl.when(p == 1)
    def _():
        v = vi_s[pl.ds(i * BN, BN), :]
        onehot = (b == sess).astype(_f32)
        vn_rep = lax.dot_general(onehot, vn_ref[...], (((1,), (0,)), ((), ())),
                                 preferred_element_type=_f32)
        q1 = _dotT(vn_rep, w1_ref[...])
        q2 = _dotT(v, w2_ref[...]) + b2_ref[...]
        a = jax.nn.sigmoid(q1 + q2)
        alpha = jnp.sum(a * qw_ref[...], axis=1, keepdims=True) + qb_ref[0, 0]
        contrib = lax.dot_general(onehot, alpha * v, (((0,), (0,)), ((), ())),
                                  preferred_element_type=_f32)

        @pl.when(i == 0)
        def _():
            sg_ref[...] = jnp.zeros_like(sg_ref)

        sg_ref[...] += contrib


def _tc_node(ma, mb, emb, w_ih, w_hh, batch2, bnext2, b00, w1, w2, b2r, qwr, qbr):
    with _jax_config.enable_x64(False):
        return _tc_node_x32(ma, mb, emb, w_ih, w_hh, batch2, bnext2, b00,
                            w1, w2, b2r, qwr, qbr)


def _tc_node_x32(ma, mb, emb, w_ih, w_hh, batch2, bnext2, b00, w1, w2, b2r,
                 qwr, qbr):
    grid = (2, N_NODES // BN)
    blk0 = pl.BlockSpec((BN, HIDDEN), lambda p, i: (i * (1 - p), 0))
    full = lambda r, c: pl.BlockSpec((r, c), lambda p, i: (0, 0))
    sess_spec = pl.BlockSpec((N_SESSIONS, HIDDEN), lambda p, i: (0, 0))
    return pl.pallas_call(
        _node_body,
        grid=grid,
        in_specs=[
            blk0, blk0, blk0,
            full(3 * HIDDEN, HIDDEN), full(3 * HIDDEN, HIDDEN),
            pl.BlockSpec((BN, 1), lambda p, i: (i, 0)),
            pl.BlockSpec((BN, 1), lambda p, i: (i * (1 - p), 0)),
            full(1, 1),
            full(HIDDEN, HIDDEN), full(HIDDEN, HIDDEN),
            full(1, HIDDEN), full(1, HIDDEN), full(1, 1),
        ],
        out_specs=[sess_spec, sess_spec],
        out_shape=[
            jax.ShapeDtypeStruct((N_SESSIONS, HIDDEN), _f32),
            jax.ShapeDtypeStruct((N_SESSIONS, HIDDEN), _f32),
        ],
        scratch_shapes=[pltpu.VMEM((N_NODES, HIDDEN), _f32)],
    )(ma, mb, emb, w_ih, w_hh, batch2, bnext2, b00, w1, w2, b2r, qwr, qbr)


def _vocab_body(vn_ref, sg_ref, w3_ref, wc_ref, wt_ref, wo_ref,
                emb_ref, z1_ref, z2_ref, z3_ref, xs_ref):
    @pl.when(pl.program_id(0) == 0)
    def _():
        sh = (_dotT(vn_ref[...], w3_ref[:, :HIDDEN])
              + _dotT(sg_ref[...], w3_ref[:, HIDDEN:]))
        sig = jax.nn.sigmoid(sh)
        xs_ref[:N_SESSIONS, :] = _dotT(sig, wc_ref[...])
        xs_ref[N_SESSIONS:2 * N_SESSIONS, :] = _dotT(sig, wt_ref[...])
        xs_ref[2 * N_SESSIONS:, :] = _dotT(sig, wo_ref[...])

    zz = _dotT(xs_ref[...], emb_ref[...])
    z1_ref[...] = zz[:N_SESSIONS]
    z2_ref[...] = zz[N_SESSIONS:2 * N_SESSIONS]
    z3_ref[...] = zz[2 * N_SESSIONS:]


def _tc_vocab(vn, sg, w3, wc, wt, wo, emb_weight):
    with _jax_config.enable_x64(False):
        return _tc_vocab_x32(vn, sg, w3, wc, wt, wo, emb_weight)


def _tc_vocab_x32(vn, sg, w3, wc, wt, wo, emb_weight):
    grid = (pl.cdiv(N_ITEMS, BV),)
    full = lambda r, c: pl.BlockSpec((r, c), lambda i: (0, 0))
    return pl.pallas_call(
        _vocab_body,
        grid=grid,
        in_specs=[
            full(N_SESSIONS, HIDDEN), full(N_SESSIONS, HIDDEN),
            full(HIDDEN, 2 * HIDDEN),
            full(HIDDEN, HIDDEN), full(HIDDEN, HIDDEN), full(HIDDEN, HIDDEN),
            pl.BlockSpec((BV, HIDDEN), lambda i: (i, 0)),
        ],
        out_specs=[pl.BlockSpec((N_SESSIONS, BV), lambda i: (0, i))] * 3,
        out_shape=[jax.ShapeDtypeStruct((N_SESSIONS, N_ITEMS), _f32)] * 3,
        scratch_shapes=[pltpu.VMEM((3 * N_SESSIONS, HIDDEN), _f32)],
    )(vn, sg, w3, wc, wt, wo, emb_weight)


def kernel(x, edge_index, batch, emb_weight, gru_w_ih, gru_w_hh, w1, w2, b2,
           q_w, q_b, w3, w_clicks, w_carts, w_orders):
    x32 = x.astype(jnp.int32)
    src = edge_index[0].astype(jnp.int32)
    dst = edge_index[1].astype(jnp.int32)
    batch32 = batch.astype(jnp.int32)

    x2d = jnp.pad(x32, (0, NODE_PAD - N_NODES)).reshape(NODE_PAD // GCH, GCH)
    emb_pad = _sc_embedding_gather(emb_weight, x2d)
    emb = emb_pad[:N_NODES]

    zeros = jnp.zeros((ROWS_PER_SUB + 16, HIDDEN), _f32)
    m_part = _sc_edge_scatter(emb_pad, src, dst, zeros)

    batch2 = batch32.reshape(N_NODES, 1)
    bnext2 = jnp.concatenate(
        [batch32[1:], jnp.full((1,), N_SESSIONS, jnp.int32)]).reshape(N_NODES, 1)
    b00 = batch32[:1].reshape(1, 1)

    vn, sg = _tc_node(m_part[:N_NODES], m_part[N_NODES:], emb,
                      gru_w_ih, gru_w_hh, batch2, bnext2, b00,
                      w1, w2, b2.reshape(1, HIDDEN), q_w.reshape(1, HIDDEN),
                      q_b.reshape(1, 1))

    z1, z2, z3 = _tc_vocab(vn, sg, w3, w_clicks, w_carts, w_orders, emb_weight)
    return (z1, z2, z3)


# speedup vs baseline: 1.0872x; 1.0224x over previous
import functools

import jax
import jax.numpy as jnp
import numpy as np
from jax import lax
from jax.experimental import pallas as pl
from jax.experimental.pallas import tpu as pltpu
from jax.experimental.pallas import tpu_sc as plsc
from jax._src import config as _jax_config

N_NODES = 10000
N_EDGES = 320000
HIDDEN = 128
N_ITEMS = 100000
N_SESSIONS = 32

NC = 2
NS = 16
NW = NC * NS

NODE_PAD = 10240
GCH = 40
ECH = 128
NBUF = 2
EPW = N_EDGES // NW
ETAIL = EPW % ECH
ACC_ROWS = N_NODES
ROWS_PER_SUB = 624

BN = 400
BV = 2048

_f32 = jnp.float32


def _sc_embedding_gather(emb_weight, x_pad):
    with _jax_config.enable_x64(False):
        return _sc_embedding_gather_x32(emb_weight, x_pad)


def _sc_embedding_gather_x32(emb_weight, x2d):
    mesh = plsc.VectorSubcoreMesh(core_axis_name="c", subcore_axis_name="s")
    npw = NODE_PAD // NW
    nch = npw // GCH

    @functools.partial(
        pl.kernel,
        out_type=jax.ShapeDtypeStruct((NODE_PAD, HIDDEN), _f32),
        mesh=mesh,
        scratch_types=[
            pltpu.VMEM((nch, GCH), jnp.int32),
            pltpu.VMEM((npw, HIDDEN), _f32),
            pltpu.SemaphoreType.DMA,
        ],
    )
    def k(emb_hbm, idx_hbm, out_hbm, idx_v, rows_v, sem):
        wid = lax.axis_index("s") * jnp.int32(NC) + lax.axis_index("c")
        pltpu.sync_copy(idx_hbm.at[pl.ds(wid * jnp.int32(nch), nch)], idx_v)
        for j in range(nch):
            pltpu.async_copy(emb_hbm.at[idx_v.at[jnp.int32(j)]],
                             rows_v.at[pl.ds(jnp.int32(j * GCH), GCH)], sem)
        for j in range(nch):
            pltpu.make_async_copy(emb_hbm.at[pl.ds(jnp.int32(0), GCH)],
                                  rows_v.at[pl.ds(jnp.int32(0), GCH)], sem).wait()
        pltpu.sync_copy(rows_v, out_hbm.at[pl.ds(wid * jnp.int32(npw), npw)])

    return k(emb_weight, x2d)


def _sc_edge_scatter(emb, src, dst, zeros):
    with _jax_config.enable_x64(False):
        return _sc_edge_scatter_x32(emb, src, dst, zeros)


def _sc_edge_scatter_x32(emb, src2d, dst2d, zeros):
    mesh = plsc.VectorSubcoreMesh(core_axis_name="c", subcore_axis_name="s")
    nch = EPW // ECH
    nmain = nch - (nch % NBUF)

    @functools.partial(
        pl.kernel,
        out_type=jax.ShapeDtypeStruct((NC * N_NODES, HIDDEN), _f32),
        mesh=mesh,
        scratch_types=(
            [pltpu.VMEM((ECH,), jnp.int32)] * (2 * NBUF)
            + [pltpu.VMEM((ETAIL,), jnp.int32)] * 2
            + [pltpu.VMEM((ECH, HIDDEN), _f32)] * NBUF
            + [pltpu.VMEM_SHARED((ACC_ROWS, HIDDEN), _f32)]
            + [pltpu.SemaphoreType.DMA] * NBUF
        ),
    )
    def k(emb_hbm, src_hbm, dst_hbm, zeros_hbm, m_hbm,
          src_v0, src_v1, dst_v0, dst_v1, src_t, dst_t,
          rows0, rows1, acc, sem0, sem1):
        cid = lax.axis_index("c")
        sid = lax.axis_index("s")
        wid = sid * jnp.int32(NC) + cid
        off = sid * jnp.int32(ROWS_PER_SUB)
        srcs = (src_v0, src_v1)
        dsts = (dst_v0, dst_v1)
        rows = (rows0, rows1)
        sems = (sem0, sem1)

        pltpu.sync_copy(zeros_hbm.at[pl.ds(0, ROWS_PER_SUB)],
                        acc.at[pl.ds(off, ROWS_PER_SUB)])

        @pl.when(sid == jnp.int32(NS - 1))
        def _():
            pltpu.sync_copy(zeros_hbm.at[pl.ds(ROWS_PER_SUB, 16)],
                            acc.at[pl.ds(NS * ROWS_PER_SUB, 16)])

        plsc.subcore_barrier()

        def stage_and_gather(ch, b):
            base = wid * jnp.int32(EPW) + ch * jnp.int32(ECH)
            pltpu.sync_copy(src_hbm.at[pl.ds(base, ECH)], srcs[b])
            pltpu.sync_copy(dst_hbm.at[pl.ds(base, ECH)], dsts[b])
            pltpu.async_copy(emb_hbm.at[srcs[b]], rows[b], sems[b])

        def wait_gather(b):
            pltpu.make_async_copy(emb_hbm.at[pl.ds(jnp.int32(0), ECH)],
                                  rows[b], sems[b]).wait()

        for b in range(NBUF):
            stage_and_gather(jnp.int32(b), b)

        @pl.loop(np.int32(0), np.int32(nmain), step=np.int32(NBUF))
        def _(g):
            for b in range(NBUF):
                ch = g + jnp.int32(b)
                wait_gather(b)
                pltpu.sync_copy(rows[b], acc.at[dsts[b]], add=True)
                nxt = ch + jnp.int32(NBUF)

                @pl.when(nxt < jnp.int32(nch))
                def _():
                    stage_and_gather(nxt, b)

        tbase = wid * jnp.int32(EPW) + jnp.int32(nch * ECH)
        pltpu.sync_copy(src_hbm.at[pl.ds(tbase, ETAIL)], src_t)
        pltpu.sync_copy(dst_hbm.at[pl.ds(tbase, ETAIL)], dst_t)
        for b in range(nch - nmain):
            wait_gather(b)
            pltpu.sync_copy(rows[b], acc.at[dsts[b]], add=True)
        pltpu.async_copy(emb_hbm.at[src_t],
                         rows0.at[pl.ds(jnp.int32(0), ETAIL)], sem0).wait()
        pltpu.sync_copy(rows0.at[pl.ds(jnp.int32(0), ETAIL)],
                        acc.at[dst_t], add=True)
        plsc.subcore_barrier()

        obase = cid * jnp.int32(N_NODES)
        pltpu.sync_copy(acc.at[pl.ds(off, ROWS_PER_SUB)],
                        m_hbm.at[pl.ds(obase + off, ROWS_PER_SUB)])

        @pl.when(sid == jnp.int32(NS - 1))
        def _():
            pltpu.sync_copy(acc.at[pl.ds(NS * ROWS_PER_SUB, 16)],
                            m_hbm.at[pl.ds(obase + jnp.int32(NS * ROWS_PER_SUB), 16)])

    return k(emb, src2d, dst2d, zeros)


def _dotT(a, b):
    return lax.dot_general(a, b, (((1,), (1,)), ((), ())),
                           preferred_element_type=_f32)


def _node_body(ma_ref, mb_ref, emb_ref, wih_ref, whh_ref,
               batch_ref, bnext_ref, b0_ref, w1_ref, w2_ref, b2_ref,
               qw_ref, qb_ref, vn_ref, sg_ref, vi_s):
    p = pl.program_id(0)
    i = pl.program_id(1)
    b = batch_ref[...]
    sess = lax.broadcasted_iota(jnp.int32, (BN, N_SESSIONS), 1)

    @pl.when(p == 0)
    def _():
        m = ma_ref[...] + mb_ref[...]
        h = emb_ref[...]
        gi = _dotT(m, wih_ref[...])
        gh = _dotT(h, whh_ref[...])
        r = jax.nn.sigmoid(gi[:, :HIDDEN] + gh[:, :HIDDEN])
        z = jax.nn.sigmoid(gi[:, HIDDEN:2 * HIDDEN] + gh[:, HIDDEN:2 * HIDDEN])
        n = jnp.tanh(gi[:, 2 * HIDDEN:] + r * gh[:, 2 * HIDDEN:])
        v = (1.0 - z) * n + z * h
        vi_s[pl.ds(i * BN, BN), :] = v

        bn = bnext_ref[...]
        b0 = b0_ref[...]
        last = (b <= sess) & (bn > sess)
        i_glob = i * BN + lax.broadcasted_iota(jnp.int32, (BN, N_SESSIONS), 0)
        wrap = (i_glob == N_NODES - 1) & (sess < b0)
        onehot_last = (last | wrap).astype(_f32)
        contrib = lax.dot_general(onehot_last, v, (((0,), (0,)), ((), ())),
                                  preferred_element_type=_f32)

        @pl.when(i == 0)
        def _():
            vn_ref[...] = jnp.zeros_like(vn_ref)

        vn_ref[...] += contrib

    @pl.when(p == 1)
    def _():
        v = vi_s[pl.ds(i * BN, BN), :]
        onehot = (b == sess).astype(_f32)
        vn_rep = lax.dot_general(onehot, vn_ref[...], (((1,), (0,)), ((), ())),
                                 preferred_element_type=_f32)
        q1 = _dotT(vn_rep, w1_ref[...])
        q2 = _dotT(v, w2_ref[...]) + b2_ref[...]
        a = jax.nn.sigmoid(q1 + q2)
        alpha = jnp.sum(a * qw_ref[...], axis=1, keepdims=True) + qb_ref[0, 0]
        contrib = lax.dot_general(onehot, alpha * v, (((0,), (0,)), ((), ())),
                                  preferred_element_type=_f32)

        @pl.when(i == 0)
        def _():
            sg_ref[...] = jnp.zeros_like(sg_ref)

        sg_ref[...] += contrib


def _tc_node(ma, mb, emb, w_ih, w_hh, batch2, bnext2, b00, w1, w2, b2r, qwr, qbr):
    with _jax_config.enable_x64(False):
        return _tc_node_x32(ma, mb, emb, w_ih, w_hh, batch2, bnext2, b00,
                            w1, w2, b2r, qwr, qbr)


def _tc_node_x32(ma, mb, emb, w_ih, w_hh, batch2, bnext2, b00, w1, w2, b2r,
                 qwr, qbr):
    grid = (2, N_NODES // BN)
    blk0 = pl.BlockSpec((BN, HIDDEN), lambda p, i: (i * (1 - p), 0))
    blk1 = pl.BlockSpec((BN, HIDDEN),
                        lambda p, i: (i * (1 - p) + N_NODES // BN, 0))
    full = lambda r, c: pl.BlockSpec((r, c), lambda p, i: (0, 0))
    sess_spec = pl.BlockSpec((N_SESSIONS, HIDDEN), lambda p, i: (0, 0))
    return pl.pallas_call(
        _node_body,
        grid=grid,
        in_specs=[
            blk0, blk1, blk0,
            full(3 * HIDDEN, HIDDEN), full(3 * HIDDEN, HIDDEN),
            pl.BlockSpec((BN, 1), lambda p, i: (i, 0)),
            pl.BlockSpec((BN, 1), lambda p, i: (i * (1 - p), 0)),
            full(1, 1),
            full(HIDDEN, HIDDEN), full(HIDDEN, HIDDEN),
            full(1, HIDDEN), full(1, HIDDEN), full(1, 1),
        ],
        out_specs=[sess_spec, sess_spec],
        out_shape=[
            jax.ShapeDtypeStruct((N_SESSIONS, HIDDEN), _f32),
            jax.ShapeDtypeStruct((N_SESSIONS, HIDDEN), _f32),
        ],
        scratch_shapes=[pltpu.VMEM((N_NODES, HIDDEN), _f32)],
    )(ma, mb, emb, w_ih, w_hh, batch2, bnext2, b00, w1, w2, b2r, qwr, qbr)


def _vocab_body(vn_ref, sg_ref, w3_ref, wc_ref, wt_ref, wo_ref,
                emb_ref, z1_ref, z2_ref, z3_ref, xs_ref):
    @pl.when(pl.program_id(0) == 0)
    def _():
        sh = (_dotT(vn_ref[...], w3_ref[:, :HIDDEN])
              + _dotT(sg_ref[...], w3_ref[:, HIDDEN:]))
        sig = jax.nn.sigmoid(sh)
        xs_ref[:N_SESSIONS, :] = _dotT(sig, wc_ref[...])
        xs_ref[N_SESSIONS:2 * N_SESSIONS, :] = _dotT(sig, wt_ref[...])
        xs_ref[2 * N_SESSIONS:, :] = _dotT(sig, wo_ref[...])

    zz = _dotT(xs_ref[...], emb_ref[...])
    z1_ref[...] = zz[:N_SESSIONS]
    z2_ref[...] = zz[N_SESSIONS:2 * N_SESSIONS]
    z3_ref[...] = zz[2 * N_SESSIONS:]


def _tc_vocab(vn, sg, w3, wc, wt, wo, emb_weight):
    with _jax_config.enable_x64(False):
        return _tc_vocab_x32(vn, sg, w3, wc, wt, wo, emb_weight)


def _tc_vocab_x32(vn, sg, w3, wc, wt, wo, emb_weight):
    grid = (pl.cdiv(N_ITEMS, BV),)
    full = lambda r, c: pl.BlockSpec((r, c), lambda i: (0, 0))
    return pl.pallas_call(
        _vocab_body,
        grid=grid,
        in_specs=[
            full(N_SESSIONS, HIDDEN), full(N_SESSIONS, HIDDEN),
            full(HIDDEN, 2 * HIDDEN),
            full(HIDDEN, HIDDEN), full(HIDDEN, HIDDEN), full(HIDDEN, HIDDEN),
            pl.BlockSpec((BV, HIDDEN), lambda i: (i, 0)),
        ],
        out_specs=[pl.BlockSpec((N_SESSIONS, BV), lambda i: (0, i))] * 3,
        out_shape=[jax.ShapeDtypeStruct((N_SESSIONS, N_ITEMS), _f32)] * 3,
        scratch_shapes=[pltpu.VMEM((3 * N_SESSIONS, HIDDEN), _f32)],
    )(vn, sg, w3, wc, wt, wo, emb_weight)


def kernel(x, edge_index, batch, emb_weight, gru_w_ih, gru_w_hh, w1, w2, b2,
           q_w, q_b, w3, w_clicks, w_carts, w_orders):
    x32 = x.astype(jnp.int32)
    src = edge_index[0].astype(jnp.int32)
    dst = edge_index[1].astype(jnp.int32)
    batch32 = batch.astype(jnp.int32)

    x2d = jnp.pad(x32, (0, NODE_PAD - N_NODES)).reshape(NODE_PAD // GCH, GCH)
    emb_pad = _sc_embedding_gather(emb_weight, x2d)

    zeros = jnp.zeros((ROWS_PER_SUB + 16, HIDDEN), _f32)
    m_part = _sc_edge_scatter(emb_pad, src, dst, zeros)

    batch2 = batch32.reshape(N_NODES, 1)
    bnext2 = jnp.concatenate(
        [batch32[1:], jnp.full((1,), N_SESSIONS, jnp.int32)]).reshape(N_NODES, 1)
    b00 = batch32[:1].reshape(1, 1)

    vn, sg = _tc_node(m_part, m_part, emb_pad,
                      gru_w_ih, gru_w_hh, batch2, bnext2, b00,
                      w1, w2, b2.reshape(1, HIDDEN), q_w.reshape(1, HIDDEN),
                      q_b.reshape(1, 1))

    z1, z2, z3 = _tc_vocab(vn, sg, w3, w_clicks, w_carts, w_orders, emb_weight)
    return (z1, z2, z3)
